# baseline (speedup 1.0000x reference)
import functools

import jax
import jax.numpy as jnp
from jax import lax
from jax.experimental import pallas as pl
from jax.experimental.pallas import tpu as pltpu
from jax.experimental.pallas import tpu_sc as plsc

def _lanes(a, perm):
    dn = lax.GatherDimensionNumbers(
        offset_dims=(), collapsed_slice_dims=(0,), start_index_map=(0,))
    return lax.gather(a, perm[:, None], dn, (1,),
                      mode=lax.GatherScatterMode.PROMISE_IN_BOUNDS)


B = 16384
D = 128
L = 16
NC = 2
NS = 16
NW = NC * NS
RW = B // NW
C = 128
NCHUNK = RW // C


def _body(h_hbm, idx_hbm, t_hbm, rel_hbm, out_hbm,
          idx_v, h_v, t_v, r_v, out_v, gsem):
    wid = lax.axis_index("s") * NC + lax.axis_index("c")
    base_w = wid * RW
    iota = lax.iota(jnp.int32, L)

    for ci in range(NCHUNK):
        base = base_w + ci * C
        pltpu.sync_copy(idx_hbm.at[pl.ds(base, C)], idx_v)
        gather = pltpu.async_copy(rel_hbm.at[idx_v], r_v, gsem)
        pltpu.sync_copy(h_hbm.at[pl.ds(base, C)], h_v)
        pltpu.sync_copy(t_hbm.at[pl.ds(base, C)], t_v)
        gather.wait()

        @plsc.parallel_loop(0, C, unroll=2)
        def _row(rr):
            acc = (h_v[rr, pl.ds(0, L)] * r_v[rr, pl.ds(0, L)]
                   * t_v[rr, pl.ds(0, L)])
            for k in range(1, D // L):
                acc = acc + (h_v[rr, pl.ds(k * L, L)]
                             * r_v[rr, pl.ds(k * L, L)]
                             * t_v[rr, pl.ds(k * L, L)])
            for fold in (1, 2, 4, 8):
                acc = acc + _lanes(acc, iota ^ fold)
            plsc.store_compressed(out_v.at[pl.ds(rr, L)], acc,
                                  mask=iota == 0)

        pltpu.sync_copy(out_v.at[pl.ds(0, C)], out_hbm.at[pl.ds(base, C)])


@functools.partial(
    pl.kernel,
    out_type=jax.ShapeDtypeStruct((B,), jnp.float32),
    mesh=plsc.VectorSubcoreMesh(
        core_axis_name="c", subcore_axis_name="s",
        num_cores=NC, num_subcores=NS),
    compiler_params=pltpu.CompilerParams(needs_layout_passes=False),
    scratch_types=[
        pltpu.VMEM((C,), jnp.int32),
        pltpu.VMEM((C, D), jnp.float32),
        pltpu.VMEM((C, D), jnp.float32),
        pltpu.VMEM((C, D), jnp.float32),
        pltpu.VMEM((C + L,), jnp.float32),
        pltpu.SemaphoreType.DMA,
    ],
)
def _distmult_sc(h_hbm, idx_hbm, t_hbm, rel_hbm, out_hbm, *scratch):
    _body(h_hbm, idx_hbm, t_hbm, rel_hbm, out_hbm, *scratch)


def kernel(h_emb, r_idx, t_emb, rel_emb):
    return _distmult_sc(h_emb, r_idx.astype(jnp.int32), t_emb, rel_emb)


# speedup vs baseline: 1.2265x; 1.2265x over previous
import functools

import jax
import jax.numpy as jnp
from jax import lax
from jax.experimental import pallas as pl
from jax.experimental.pallas import tpu as pltpu
from jax.experimental.pallas import tpu_sc as plsc

def _lanes(a, perm):
    dn = lax.GatherDimensionNumbers(
        offset_dims=(), collapsed_slice_dims=(0,), start_index_map=(0,))
    return lax.gather(a, perm[:, None], dn, (1,),
                      mode=lax.GatherScatterMode.PROMISE_IN_BOUNDS)


B = 16384
D = 128
L = 16
NC = 2
NS = 16
NW = NC * NS
RW = B // NW
C = 128
NCHUNK = RW // C


def _body(h_hbm, idx_hbm, t_hbm, rel_hbm, out_hbm, *scratch):
    wid = lax.axis_index("s") * NC + lax.axis_index("c")
    base_w = wid * RW
    iota = lax.iota(jnp.int32, L)
    bufs = [scratch[0:9], scratch[9:18]]

    def start_in(ci, bb):
        idx_v, h_v, t_v, r_v = bb[0], bb[1], bb[2], bb[3]
        gsem, hsem, tsem = bb[5], bb[6], bb[7]
        base = base_w + ci * C
        pltpu.sync_copy(idx_hbm.at[pl.ds(base, C)], idx_v)
        g = pltpu.async_copy(rel_hbm.at[idx_v], r_v, gsem)
        h = pltpu.async_copy(h_hbm.at[pl.ds(base, C)], h_v, hsem)
        t = pltpu.async_copy(t_hbm.at[pl.ds(base, C)], t_v, tsem)
        return (g, h, t)

    pend_in = {0: start_in(0, bufs[0])}
    pend_out = {}
    for ci in range(NCHUNK):
        bb = bufs[ci & 1]
        h_v, t_v, r_v, out_v, osem = bb[1], bb[2], bb[3], bb[4], bb[8]
        if ci + 1 < NCHUNK:
            pend_in[ci + 1] = start_in(ci + 1, bufs[(ci + 1) & 1])
        for cpy in pend_in.pop(ci):
            cpy.wait()
        if ci - 2 in pend_out:
            pend_out.pop(ci - 2).wait()

        @plsc.parallel_loop(0, C, unroll=2)
        def _row(rr):
            acc = (h_v[rr, pl.ds(0, L)] * r_v[rr, pl.ds(0, L)]
                   * t_v[rr, pl.ds(0, L)])
            for k in range(1, D // L):
                acc = acc + (h_v[rr, pl.ds(k * L, L)]
                             * r_v[rr, pl.ds(k * L, L)]
                             * t_v[rr, pl.ds(k * L, L)])
            for fold in (1, 2, 4, 8):
                acc = acc + _lanes(acc, iota ^ fold)
            plsc.store_compressed(out_v.at[pl.ds(rr, L)], acc,
                                  mask=iota == 0)

        base = base_w + ci * C
        pend_out[ci] = pltpu.async_copy(
            out_v.at[pl.ds(0, C)], out_hbm.at[pl.ds(base, C)], osem)
    for cpy in pend_out.values():
        cpy.wait()


@functools.partial(
    pl.kernel,
    out_type=jax.ShapeDtypeStruct((B,), jnp.float32),
    mesh=plsc.VectorSubcoreMesh(
        core_axis_name="c", subcore_axis_name="s",
        num_cores=NC, num_subcores=NS),
    compiler_params=pltpu.CompilerParams(needs_layout_passes=False),
    scratch_types=[
        pltpu.VMEM((C,), jnp.int32),
        pltpu.VMEM((C, D), jnp.float32),
        pltpu.VMEM((C, D), jnp.float32),
        pltpu.VMEM((C, D), jnp.float32),
        pltpu.VMEM((C + L,), jnp.float32),
        pltpu.SemaphoreType.DMA,
        pltpu.SemaphoreType.DMA,
        pltpu.SemaphoreType.DMA,
        pltpu.SemaphoreType.DMA,
    ] * 2,
)
def _distmult_sc(h_hbm, idx_hbm, t_hbm, rel_hbm, out_hbm, *scratch):
    _body(h_hbm, idx_hbm, t_hbm, rel_hbm, out_hbm, *scratch)


def kernel(h_emb, r_idx, t_emb, rel_emb):
    return _distmult_sc(h_emb, r_idx.astype(jnp.int32), t_emb, rel_emb)


# speedup vs baseline: 1.2688x; 1.0345x over previous
import functools

import jax
import jax.numpy as jnp
from jax import lax
from jax.experimental import pallas as pl
from jax.experimental.pallas import tpu as pltpu
from jax.experimental.pallas import tpu_sc as plsc

def _lanes(a, perm):
    dn = lax.GatherDimensionNumbers(
        offset_dims=(), collapsed_slice_dims=(0,), start_index_map=(0,))
    return lax.gather(a, perm[:, None], dn, (1,),
                      mode=lax.GatherScatterMode.PROMISE_IN_BOUNDS)


B = 16384
D = 128
L = 16
NC = 2
NS = 16
NW = NC * NS
RW = B // NW
C = 128
NCHUNK = RW // C
NR = 1000


def _body(h_hbm, idx_hbm, t_hbm, rel_hbm, out_hbm, *scratch):
    wid = lax.axis_index("s") * NC + lax.axis_index("c")
    base_w = wid * RW
    iota = lax.iota(jnp.int32, L)
    bufs = [scratch[0:9], scratch[9:18]]
    rel_sh = scratch[18]

    @pl.when(lax.axis_index("s") == 0)
    def _stage():
        pltpu.sync_copy(rel_hbm, rel_sh)
    plsc.subcore_barrier()

    def start_in(ci, bb):
        idx_v, h_v, t_v, r_v = bb[0], bb[1], bb[2], bb[3]
        gsem, hsem, tsem = bb[5], bb[6], bb[7]
        base = base_w + ci * C
        pltpu.sync_copy(idx_hbm.at[pl.ds(base, C)], idx_v)
        g = pltpu.async_copy(rel_sh.at[idx_v], r_v, gsem)
        h = pltpu.async_copy(h_hbm.at[pl.ds(base, C)], h_v, hsem)
        t = pltpu.async_copy(t_hbm.at[pl.ds(base, C)], t_v, tsem)
        return (g, h, t)

    pend_in = {0: start_in(0, bufs[0])}
    pend_out = {}
    for ci in range(NCHUNK):
        bb = bufs[ci & 1]
        h_v, t_v, r_v, out_v, osem = bb[1], bb[2], bb[3], bb[4], bb[8]
        if ci + 1 < NCHUNK:
            pend_in[ci + 1] = start_in(ci + 1, bufs[(ci + 1) & 1])
        for cpy in pend_in.pop(ci):
            cpy.wait()
        if ci - 2 in pend_out:
            pend_out.pop(ci - 2).wait()

        @plsc.parallel_loop(0, C, unroll=2)
        def _row(rr):
            acc = (h_v[rr, pl.ds(0, L)] * r_v[rr, pl.ds(0, L)]
                   * t_v[rr, pl.ds(0, L)])
            for k in range(1, D // L):
                acc = acc + (h_v[rr, pl.ds(k * L, L)]
                             * r_v[rr, pl.ds(k * L, L)]
                             * t_v[rr, pl.ds(k * L, L)])
            for fold in (1, 2, 4, 8):
                acc = acc + _lanes(acc, iota ^ fold)
            plsc.store_compressed(out_v.at[pl.ds(rr, L)], acc,
                                  mask=iota == 0)

        base = base_w + ci * C
        pend_out[ci] = pltpu.async_copy(
            out_v.at[pl.ds(0, C)], out_hbm.at[pl.ds(base, C)], osem)
    for cpy in pend_out.values():
        cpy.wait()


@functools.partial(
    pl.kernel,
    out_type=jax.ShapeDtypeStruct((B,), jnp.float32),
    mesh=plsc.VectorSubcoreMesh(
        core_axis_name="c", subcore_axis_name="s",
        num_cores=NC, num_subcores=NS),
    compiler_params=pltpu.CompilerParams(needs_layout_passes=False),
    scratch_types=[
        pltpu.VMEM((C,), jnp.int32),
        pltpu.VMEM((C, D), jnp.float32),
        pltpu.VMEM((C, D), jnp.float32),
        pltpu.VMEM((C, D), jnp.float32),
        pltpu.VMEM((C + L,), jnp.float32),
        pltpu.SemaphoreType.DMA,
        pltpu.SemaphoreType.DMA,
        pltpu.SemaphoreType.DMA,
        pltpu.SemaphoreType.DMA,
    ] * 2 + [pltpu.VMEM_SHARED((NR, D), jnp.float32)],
)
def _distmult_sc(h_hbm, idx_hbm, t_hbm, rel_hbm, out_hbm, *scratch):
    _body(h_hbm, idx_hbm, t_hbm, rel_hbm, out_hbm, *scratch)


def kernel(h_emb, r_idx, t_emb, rel_emb):
    return _distmult_sc(h_emb, r_idx.astype(jnp.int32), t_emb, rel_emb)
